# baseline (speedup 1.0000x reference)
import functools
import math

import jax
import jax.numpy as jnp
from jax import lax
from jax.experimental import pallas as pl
from jax.experimental.pallas import tpu as pltpu
from jax.experimental.pallas import tpu_sc as plsc

D_MODEL = 64
CHUNK = 128
SCALE = math.sqrt(D_MODEL)
LANES = 16


@functools.lru_cache(maxsize=None)
def _build(B):
    info = plsc.get_sparse_core_info()
    nc, ns = info.num_cores, info.num_subcores
    nw = nc * ns
    assert B % (nw * CHUNK) == 0
    b_per_w = B // nw
    n_chunks = b_per_w // CHUNK

    mesh = plsc.VectorSubcoreMesh(core_axis_name="c", subcore_axis_name="s")

    @functools.partial(
        pl.kernel,
        mesh=mesh,
        compiler_params=pltpu.CompilerParams(use_tc_tiling_on_sc=False),
        out_type=jax.ShapeDtypeStruct((B, D_MODEL), jnp.float32),
        scratch_types=[
            pltpu.VMEM((n_chunks, CHUNK), jnp.int32),
            pltpu.VMEM((CHUNK, D_MODEL), jnp.float32),
            pltpu.SemaphoreType.DMA,
        ],
    )
    def emb(idx_hbm, table_hbm, out_hbm, idx_v, rows_v, sem):
        wid = lax.axis_index("s") * nc + lax.axis_index("c")
        pltpu.sync_copy(idx_hbm.at[wid], idx_v)
        base = wid * b_per_w

        def chunk_body(c, _):
            pltpu.async_copy(table_hbm.at[idx_v.at[c]], rows_v, sem).wait()

            def row_body(i, _):
                for j in range(D_MODEL // LANES):
                    sl = pl.ds(j * LANES, LANES)
                    rows_v[i, sl] = rows_v[i, sl] * SCALE
                return 0

            lax.fori_loop(0, CHUNK, row_body, 0)
            pltpu.sync_copy(rows_v, out_hbm.at[pl.ds(base + c * CHUNK, CHUNK)])
            return 0

        lax.fori_loop(0, n_chunks, chunk_body, 0)

    return emb, nw, n_chunks


def kernel(x, table):
    S0, S1 = x.shape
    B = S0 * S1
    emb, nw, n_chunks = _build(B)
    idx = x.reshape(nw, n_chunks, CHUNK).astype(jnp.int32)
    out = emb(idx, table)
    return out.reshape(S0, S1, D_MODEL)


# speedup vs baseline: 1.2024x; 1.2024x over previous
import functools
import math

import jax
import jax.numpy as jnp
from jax import lax
from jax.experimental import pallas as pl
from jax.experimental.pallas import tpu as pltpu
from jax.experimental.pallas import tpu_sc as plsc

D_MODEL = 64
CHUNK = 128
NBUF = 4
LEAD = 2
SCALE = math.sqrt(D_MODEL)
LANES = 16
ROW_UNROLL = 8


@functools.lru_cache(maxsize=None)
def _build(B):
    info = plsc.get_sparse_core_info()
    nc, ns = info.num_cores, info.num_subcores
    nw = nc * ns
    assert B % (nw * CHUNK) == 0
    b_per_w = B // nw
    n_chunks = b_per_w // CHUNK
    assert n_chunks % NBUF == 0 and n_chunks > NBUF

    mesh = plsc.VectorSubcoreMesh(core_axis_name="c", subcore_axis_name="s")

    @functools.partial(
        pl.kernel,
        mesh=mesh,
        compiler_params=pltpu.CompilerParams(use_tc_tiling_on_sc=False),
        out_type=jax.ShapeDtypeStruct((B, D_MODEL), jnp.float32),
        scratch_types=(
            [pltpu.VMEM((n_chunks, CHUNK), jnp.int32)]
            + [pltpu.VMEM((CHUNK, D_MODEL), jnp.float32) for _ in range(NBUF)]
            + [pltpu.SemaphoreType.DMA for _ in range(2 * NBUF)]
        ),
    )
    def emb(idx_hbm, table_hbm, out_hbm, idx_v, *bufs_and_sems):
        bufs = bufs_and_sems[:NBUF]
        g_sem = bufs_and_sems[NBUF:2 * NBUF]
        s_sem = bufs_and_sems[2 * NBUF:]
        wid = lax.axis_index("s") * nc + lax.axis_index("c")
        base = wid * b_per_w
        pltpu.sync_copy(idx_hbm.at[wid], idx_v)

        def start_gather(c, b):
            pltpu.async_copy(table_hbm.at[idx_v.at[c]], bufs[b], g_sem[b])

        def wait_gather(c, b):
            pltpu.make_async_copy(
                table_hbm.at[idx_v.at[c]], bufs[b], g_sem[b]).wait()

        def out_slice(c):
            return out_hbm.at[pl.ds(base + c * CHUNK, CHUNK)]

        def start_store(c, b):
            pltpu.async_copy(bufs[b], out_slice(c), s_sem[b])

        def wait_store(c, b):
            pltpu.make_async_copy(bufs[b], out_slice(c), s_sem[b]).wait()

        for b in range(LEAD):
            start_gather(b, b)

        def outer(o, _):
            c0 = o * NBUF
            for b in range(NBUF):
                c = c0 + b
                j = c + LEAD
                bj = (b + LEAD) % NBUF

                @pl.when(jnp.logical_and(j < n_chunks, j >= NBUF))
                def _():
                    wait_store(j - NBUF, bj)

                @pl.when(j < n_chunks)
                def _():
                    start_gather(j, bj)

                wait_gather(c, b)

                def row_body(i, _):
                    r0 = i * ROW_UNROLL
                    for r in range(ROW_UNROLL):
                        for j4 in range(D_MODEL // LANES):
                            sl = pl.ds(j4 * LANES, LANES)
                            bufs[b][r0 + r, sl] = bufs[b][r0 + r, sl] * SCALE
                    return 0

                lax.fori_loop(0, CHUNK // ROW_UNROLL, row_body, 0,
                              unroll=False)
                start_store(c, b)
            return 0

        lax.fori_loop(0, n_chunks // NBUF, outer, 0, unroll=False)

        for b in range(NBUF):
            wait_store(n_chunks - NBUF + b, b)

    return emb, nw, n_chunks


def kernel(x, table):
    S0, S1 = x.shape
    B = S0 * S1
    emb, nw, n_chunks = _build(B)
    idx = x.reshape(nw, n_chunks, CHUNK).astype(jnp.int32)
    out = emb(idx, table)
    return out.reshape(S0, S1, D_MODEL)
